# baseline (speedup 1.0000x reference)
import jax
import jax.numpy as jnp
from jax.experimental import pallas as pl
from jax.experimental.pallas import tpu as pltpu


def _body(s_ref, l_ref, o_ref, e_ref, out_ref, acc_ref):
    i = pl.program_id(0)
    j = pl.program_id(1)
    gi = pl.num_programs(0)
    gj = pl.num_programs(1)

    @pl.when((i == 0) & (j == 0))
    def _():
        acc_ref[0] = 0.0
        acc_ref[1] = 0.0
        acc_ref[2] = 0.0

    lab = l_ref[...]
    fg = lab != 0
    nfg_part = jnp.sum(fg.astype(jnp.float32))

    d = o_ref[...] - e_ref[...]
    ad = jnp.abs(d)
    sl1 = jnp.where(ad < 1.0, 0.5 * d * d, ad - 0.5)
    sl1_part = jnp.sum(jnp.where(fg[:, None, :], sl1, 0.0))

    labT = jnp.swapaxes(lab, 0, 1)

    ce_part = 0.0
    R = labT.shape[0]
    C = s_ref.shape[2]
    iota = jax.lax.broadcasted_iota(jnp.int32, (R, C), 1)
    for bb in range(8):
        s = s_ref[bb]
        m = jnp.max(s, axis=1, keepdims=True)
        lse = m + jnp.log(jnp.sum(jnp.exp(s - m), axis=1, keepdims=True))
        lab_col = labT[:, bb : bb + 1]
        picked = jnp.sum(jnp.where(iota == lab_col, s, 0.0), axis=1,
                         keepdims=True)
        ce_part += jnp.sum(lse - picked)

    acc_ref[0] += ce_part
    acc_ref[1] += nfg_part
    acc_ref[2] += sl1_part

    @pl.when((i == gi - 1) & (j == gj - 1))
    def _():
        nf = acc_ref[1]
        cls = acc_ref[0] / nf
        reg = acc_ref[2] / nf
        out_ref[0] = cls
        out_ref[1] = reg
        out_ref[2] = cls + reg


def kernel(offsets, scores, assigned_labels, encoded_bboxes):
    B, A, C = scores.shape
    R = 1024
    GB = B // 8
    GA = A // R

    lab2 = assigned_labels.reshape(B, A)
    off3 = jnp.swapaxes(offsets, 1, 2)
    enc3 = jnp.swapaxes(encoded_bboxes, 1, 2)

    out = pl.pallas_call(
        _body,
        grid=(GB, GA),
        in_specs=[
            pl.BlockSpec((8, R, C), lambda i, j: (i, j, 0)),
            pl.BlockSpec((8, R), lambda i, j: (i, j)),
            pl.BlockSpec((8, 4, R), lambda i, j: (i, 0, j)),
            pl.BlockSpec((8, 4, R), lambda i, j: (i, 0, j)),
        ],
        out_specs=pl.BlockSpec(memory_space=pltpu.SMEM),
        out_shape=jax.ShapeDtypeStruct((3,), jnp.float32),
        scratch_shapes=[pltpu.SMEM((3,), jnp.float32)],
    )(scores, lab2, off3, enc3)

    return {
        "total_loss": out[2],
        "regre_loss": out[1],
        "classification_loss": out[0],
    }
